# baseline (speedup 1.0000x reference)
import jax
import jax.numpy as jnp
from jax.experimental import pallas as pl
from jax.experimental.pallas import tpu as pltpu

D_MODEL = 128
SEQ = 64
BATCH = 4096
CHUNK_ROWS = 64
NCHUNKS = BATCH // CHUNK_ROWS
NBUF = 4
LOOKAHEAD = 2
OUTER = NCHUNKS // NBUF


def _body(inds_ref, x_hbm, pe_hbm, o_hbm, pe_v, fp, bufs, rsems, wsems, gsem):
    def read(c, slot):
        return pltpu.make_async_copy(
            x_hbm.at[pl.ds(c * CHUNK_ROWS, CHUNK_ROWS)],
            bufs.at[slot], rsems.at[slot])

    def write(c, slot):
        return pltpu.make_async_copy(
            bufs.at[slot],
            o_hbm.at[pl.ds(c * CHUNK_ROWS, CHUNK_ROWS)], wsems.at[slot])

    pltpu.make_async_copy(pe_hbm, pe_v, gsem).start()
    read(0, 0).start()
    read(1, 1).start()
    pltpu.make_async_copy(pe_hbm, pe_v, gsem).wait()

    def gather_row(j, _):
        idx = inds_ref[j]
        fp[pl.ds(j, 1), :] = pe_v[pl.ds(idx, 1), :]
        return 0

    jax.lax.fori_loop(0, SEQ, gather_row, 0)

    def outer(i2, _):
        for b in range(NBUF):
            c = i2 * NBUF + b
            s2 = (b + LOOKAHEAD) % NBUF
            c2 = c + LOOKAHEAD

            @pl.when(c2 < NCHUNKS)
            def _():
                @pl.when(c >= LOOKAHEAD)
                def _():
                    write(jnp.maximum(c - LOOKAHEAD, 0), s2).wait()

                read(c2, s2).start()

            read(c, b).wait()
            bufs[b] = bufs[b] + fp[...][None, :, :]
            write(c, b).start()
        return 0

    jax.lax.fori_loop(0, OUTER, outer, 0)

    for c in range(NCHUNKS - NBUF, NCHUNKS):
        write(c, c % NBUF).wait()


def kernel(x, x_node_inds, pe):
    inds = x_node_inds.astype(jnp.int32)
    pe64 = pe[:SEQ]

    grid_spec = pltpu.PrefetchScalarGridSpec(
        num_scalar_prefetch=1,
        grid=(1,),
        in_specs=[
            pl.BlockSpec(memory_space=pl.ANY),
            pl.BlockSpec(memory_space=pl.ANY),
        ],
        out_specs=pl.BlockSpec(memory_space=pl.ANY),
        scratch_shapes=[
            pltpu.VMEM((SEQ, D_MODEL), jnp.float32),
            pltpu.VMEM((SEQ, D_MODEL), jnp.float32),
            pltpu.VMEM((NBUF, CHUNK_ROWS, SEQ, D_MODEL), jnp.float32),
            pltpu.SemaphoreType.DMA((NBUF,)),
            pltpu.SemaphoreType.DMA((NBUF,)),
            pltpu.SemaphoreType.DMA,
        ],
    )

    return pl.pallas_call(
        _body,
        grid_spec=grid_spec,
        out_shape=jax.ShapeDtypeStruct(x.shape, x.dtype),
        compiler_params=pltpu.CompilerParams(
            dimension_semantics=("arbitrary",),
        ),
    )(inds, x, pe64)


# speedup vs baseline: 1.0250x; 1.0250x over previous
import jax
import jax.numpy as jnp
from jax.experimental import pallas as pl
from jax.experimental.pallas import tpu as pltpu

D_MODEL = 128
SEQ = 64
BATCH = 4096
CHUNK_ROWS = 128
NCHUNKS = BATCH // CHUNK_ROWS
NBUF = 4
LOOKAHEAD = 2
OUTER = NCHUNKS // NBUF


def _body(inds_ref, x_hbm, pe_hbm, o_hbm, pe_v, fp, bufs, rsems, wsems, gsem):
    def read(c, slot):
        return pltpu.make_async_copy(
            x_hbm.at[pl.ds(c * CHUNK_ROWS, CHUNK_ROWS)],
            bufs.at[slot], rsems.at[slot])

    def write(c, slot):
        return pltpu.make_async_copy(
            bufs.at[slot],
            o_hbm.at[pl.ds(c * CHUNK_ROWS, CHUNK_ROWS)], wsems.at[slot])

    pltpu.make_async_copy(pe_hbm, pe_v, gsem).start()
    read(0, 0).start()
    read(1, 1).start()
    pltpu.make_async_copy(pe_hbm, pe_v, gsem).wait()

    def gather_row(j, _):
        idx = inds_ref[j]
        fp[pl.ds(j, 1), :] = pe_v[pl.ds(idx, 1), :]
        return 0

    jax.lax.fori_loop(0, SEQ, gather_row, 0)

    def outer(i2, _):
        for b in range(NBUF):
            c = i2 * NBUF + b
            s2 = (b + LOOKAHEAD) % NBUF
            c2 = c + LOOKAHEAD

            @pl.when(c2 < NCHUNKS)
            def _():
                @pl.when(c >= LOOKAHEAD)
                def _():
                    write(jnp.maximum(c - LOOKAHEAD, 0), s2).wait()

                read(c2, s2).start()

            read(c, b).wait()
            bufs[b] = bufs[b] + fp[...][None, :, :]
            write(c, b).start()
        return 0

    jax.lax.fori_loop(0, OUTER, outer, 0)

    for c in range(NCHUNKS - NBUF, NCHUNKS):
        write(c, c % NBUF).wait()


def kernel(x, x_node_inds, pe):
    inds = x_node_inds.astype(jnp.int32)
    pe64 = pe[:SEQ]

    grid_spec = pltpu.PrefetchScalarGridSpec(
        num_scalar_prefetch=1,
        grid=(1,),
        in_specs=[
            pl.BlockSpec(memory_space=pl.ANY),
            pl.BlockSpec(memory_space=pl.ANY),
        ],
        out_specs=pl.BlockSpec(memory_space=pl.ANY),
        scratch_shapes=[
            pltpu.VMEM((SEQ, D_MODEL), jnp.float32),
            pltpu.VMEM((SEQ, D_MODEL), jnp.float32),
            pltpu.VMEM((NBUF, CHUNK_ROWS, SEQ, D_MODEL), jnp.float32),
            pltpu.SemaphoreType.DMA((NBUF,)),
            pltpu.SemaphoreType.DMA((NBUF,)),
            pltpu.SemaphoreType.DMA,
        ],
    )

    return pl.pallas_call(
        _body,
        grid_spec=grid_spec,
        out_shape=jax.ShapeDtypeStruct(x.shape, x.dtype),
        compiler_params=pltpu.CompilerParams(
            dimension_semantics=("arbitrary",),
        ),
    )(inds, x, pe64)
